# initial kernel scaffold (unmeasured)
import jax
import jax.numpy as jnp
from jax import lax
from jax.experimental import pallas as pl
from jax.experimental.pallas import tpu as pltpu

VOCAB_PER_X = 8192


def _exchange_add(partial):
    t, d = partial.shape

    def body(p_ref, out_ref, recv_ref, send_sem, recv_sem):
        my_x = lax.axis_index("x")
        my_y = lax.axis_index("y")
        my_z = lax.axis_index("z")
        other_x = 1 - my_x

        barrier_sem = pltpu.get_barrier_semaphore()
        pl.semaphore_signal(
            barrier_sem,
            inc=1,
            device_id=(other_x, my_y, my_z),
            device_id_type=pl.DeviceIdType.MESH,
        )
        pl.semaphore_wait(barrier_sem, 1)

        rdma = pltpu.make_async_remote_copy(
            src_ref=p_ref,
            dst_ref=recv_ref,
            send_sem=send_sem,
            recv_sem=recv_sem,
            device_id=(other_x, my_y, my_z),
            device_id_type=pl.DeviceIdType.MESH,
        )
        rdma.start()
        rdma.wait()
        out_ref[...] = p_ref[...] + recv_ref[...]

    return pl.pallas_call(
        body,
        out_shape=jax.ShapeDtypeStruct((t, d), partial.dtype),
        in_specs=[pl.BlockSpec(memory_space=pltpu.VMEM)],
        out_specs=pl.BlockSpec(memory_space=pltpu.VMEM),
        scratch_shapes=[
            pltpu.VMEM((t, d), partial.dtype),
            pltpu.SemaphoreType.DMA,
            pltpu.SemaphoreType.DMA,
        ],
        compiler_params=pltpu.CompilerParams(collective_id=0),
    )(partial)


def kernel(ids, E):
    my_x = lax.axis_index("x")
    idx = ids.astype(jnp.int32) - my_x * VOCAB_PER_X
    owned = (idx >= 0) & (idx < VOCAB_PER_X)
    safe = jnp.clip(idx, 0, VOCAB_PER_X - 1)
    partial = jnp.where(owned[:, None], E[safe], jnp.float32(0.0))
    return _exchange_add(partial)


# baseline (device time: 109192 ns/iter reference)
import jax
import jax.numpy as jnp
from jax import lax
from jax.experimental import pallas as pl
from jax.experimental.pallas import tpu as pltpu

VOCAB_PER_X = 8192


def _exchange_add(partial):
    t, d = partial.shape

    def body(p_ref, out_ref, recv_ref, send_sem, recv_sem):
        my_x = lax.axis_index("x")
        my_y = lax.axis_index("y")
        my_z = lax.axis_index("z")
        other_x = 1 - my_x

        rdma = pltpu.make_async_remote_copy(
            src_ref=p_ref,
            dst_ref=recv_ref,
            send_sem=send_sem,
            recv_sem=recv_sem,
            device_id=(other_x, my_y, my_z),
            device_id_type=pl.DeviceIdType.MESH,
        )
        rdma.start()
        rdma.wait()
        out_ref[...] = p_ref[...] + recv_ref[...]

    return pl.pallas_call(
        body,
        out_shape=jax.ShapeDtypeStruct((t, d), partial.dtype),
        in_specs=[pl.BlockSpec(memory_space=pltpu.VMEM)],
        out_specs=pl.BlockSpec(memory_space=pltpu.VMEM),
        scratch_shapes=[
            pltpu.VMEM((t, d), partial.dtype),
            pltpu.SemaphoreType.DMA,
            pltpu.SemaphoreType.DMA,
        ],
    )(partial)


def kernel(ids, E):
    my_x = lax.axis_index("x")
    idx = ids.astype(jnp.int32) - my_x * VOCAB_PER_X
    owned = (idx >= 0) & (idx < VOCAB_PER_X)
    safe = jnp.clip(idx, 0, VOCAB_PER_X - 1)
    partial = jnp.where(owned[:, None], E[safe], jnp.float32(0.0))
    return _exchange_add(partial)


# device time: 60397 ns/iter; 1.8079x vs baseline; 1.8079x over previous
import jax
import jax.numpy as jnp
from jax import lax
from jax.experimental import pallas as pl
from jax.experimental.pallas import tpu as pltpu

VOCAB_PER_X = 8192
NSEM = 16


def kernel(ids, E):
    t_tokens = ids.shape[0]
    d = E.shape[1]

    def body(ids_ref, e_ref, out_ref, gsems, ssems, recv_sem):
        my_x = lax.axis_index("x")
        my_y = lax.axis_index("y")
        my_z = lax.axis_index("z")
        peer = (1 - my_x, my_y, my_z)
        base = my_x * VOCAB_PER_X

        def remote_desc(src, dst, slot):
            return pltpu.make_async_remote_copy(
                src_ref=src,
                dst_ref=dst,
                send_sem=ssems.at[slot],
                recv_sem=recv_sem,
                device_id=peer,
                device_id_type=pl.DeviceIdType.MESH,
            )

        def step(t, pos):
            idx = ids_ref[t] - base
            owned = jnp.logical_and(idx >= 0, idx < VOCAB_PER_X)

            @pl.when(owned)
            def _():
                slot = lax.rem(pos, NSEM)

                @pl.when(pos >= NSEM)
                def _():
                    pltpu.make_async_copy(
                        e_ref.at[0], out_ref.at[0], gsems.at[slot]
                    ).wait()
                    remote_desc(e_ref.at[0], out_ref.at[0], slot).wait_send()

                pltpu.make_async_copy(
                    e_ref.at[idx], out_ref.at[t], gsems.at[slot]
                ).start()
                remote_desc(e_ref.at[idx], out_ref.at[t], slot).start()

            return pos + owned.astype(jnp.int32)

        count = lax.fori_loop(0, t_tokens, step, jnp.int32(0))

        def drain(s, c):
            @pl.when(s < jnp.minimum(count, NSEM))
            def _():
                pltpu.make_async_copy(
                    e_ref.at[0], out_ref.at[0], gsems.at[s]
                ).wait()
                remote_desc(e_ref.at[0], out_ref.at[0], s).wait_send()

            return c

        lax.fori_loop(0, NSEM, drain, 0)

        n_recv = t_tokens - count

        def rwait(j, c):
            @pl.when(j < n_recv)
            def _():
                remote_desc(e_ref.at[0], out_ref.at[0], 0).wait_recv()

            return c

        lax.fori_loop(0, t_tokens, rwait, 0)

    return pl.pallas_call(
        body,
        out_shape=jax.ShapeDtypeStruct((t_tokens, d), jnp.float32),
        in_specs=[
            pl.BlockSpec(memory_space=pltpu.SMEM),
            pl.BlockSpec(memory_space=pltpu.HBM),
        ],
        out_specs=pl.BlockSpec(memory_space=pltpu.VMEM),
        scratch_shapes=[
            pltpu.SemaphoreType.DMA((NSEM,)),
            pltpu.SemaphoreType.DMA((NSEM,)),
            pltpu.SemaphoreType.DMA,
        ],
    )(ids.astype(jnp.int32), E)


# device time: 49174 ns/iter; 2.2205x vs baseline; 1.2282x over previous
import jax
import jax.numpy as jnp
from jax import lax
from jax.experimental import pallas as pl
from jax.experimental.pallas import tpu as pltpu

VOCAB_PER_X = 8192
LOG2_MAX = 11


def kernel(ids, E):
    t_tokens = ids.shape[0]
    d = E.shape[1]

    def body(count_ref, tok_ref, row_ref, e_ref, out_ref, gsem, ssem, rsem):
        my_x = lax.axis_index("x")
        my_y = lax.axis_index("y")
        my_z = lax.axis_index("z")
        peer = (1 - my_x, my_y, my_z)
        count = count_ref[0]
        n_recv = t_tokens - count

        def remote_desc(src, dst):
            return pltpu.make_async_remote_copy(
                src_ref=src,
                dst_ref=dst,
                send_sem=ssem,
                recv_sem=rsem,
                device_id=peer,
                device_id_type=pl.DeviceIdType.MESH,
            )

        def step(p, c):
            t = tok_ref[p]
            r = row_ref[p]
            pltpu.make_async_copy(e_ref.at[r], out_ref.at[t], gsem).start()
            remote_desc(e_ref.at[r], out_ref.at[t]).start()
            return c

        lax.fori_loop(0, count, step, 0)

        def bulk(n, wait_one):
            for b in range(LOG2_MAX - 1, -1, -1):
                sz = 1 << b

                @pl.when((n & sz) != 0)
                def _():
                    wait_one(sz)

        bulk(count, lambda sz: pltpu.make_async_copy(
            e_ref.at[pl.ds(0, sz), :], out_ref.at[pl.ds(0, sz), :], gsem
        ).wait())
        bulk(count, lambda sz: remote_desc(
            e_ref.at[pl.ds(0, sz), :], out_ref.at[pl.ds(0, sz), :]
        ).wait_send())
        bulk(n_recv, lambda sz: remote_desc(
            e_ref.at[pl.ds(0, sz), :], out_ref.at[pl.ds(0, sz), :]
        ).wait_recv())

    my_x = lax.axis_index("x")
    base = my_x * VOCAB_PER_X
    idx = ids.astype(jnp.int32) - base
    owned = (idx >= 0) & (idx < VOCAB_PER_X)
    pos = jnp.cumsum(owned.astype(jnp.int32)) - 1
    scatter_to = jnp.where(owned, pos, t_tokens)
    iota = jnp.arange(t_tokens, dtype=jnp.int32)
    tok = jnp.zeros((t_tokens,), jnp.int32).at[scatter_to].set(iota, mode="drop")
    row = jnp.zeros((t_tokens,), jnp.int32).at[scatter_to].set(idx, mode="drop")
    count = jnp.sum(owned.astype(jnp.int32)).reshape((1,))

    return pl.pallas_call(
        body,
        out_shape=jax.ShapeDtypeStruct((t_tokens, d), jnp.float32),
        in_specs=[
            pl.BlockSpec(memory_space=pltpu.SMEM),
            pl.BlockSpec(memory_space=pltpu.SMEM),
            pl.BlockSpec(memory_space=pltpu.SMEM),
            pl.BlockSpec(memory_space=pltpu.HBM),
        ],
        out_specs=pl.BlockSpec(memory_space=pltpu.VMEM),
        scratch_shapes=[
            pltpu.SemaphoreType.DMA,
            pltpu.SemaphoreType.DMA,
            pltpu.SemaphoreType.DMA,
        ],
    )(count, tok, row, E)
